# baseline (speedup 1.0000x reference)
import functools

import jax
import jax.numpy as jnp
from jax.experimental import pallas as pl

_T = 2048
_H = 1024
_FF = 2048
_E = 8
_TOPK = 2
_FT = 512

_dot = functools.partial(
    jax.lax.dot, precision=jax.lax.Precision.DEFAULT,
    preferred_element_type=jnp.float32)


def _moe_kernel(sel_ref, rw_ref, h_ref, w1_ref, w2_ref, w3_ref,
                out_ref, amax_ref):
    e = pl.program_id(0)
    f = pl.program_id(1)
    first = (e == 0) & (f == 0)

    cw = (jnp.where(sel_ref[:, 0:1] == e, rw_ref[:, 0:1], 0.0)
          + jnp.where(sel_ref[:, 1:2] == e, rw_ref[:, 1:2], 0.0))

    hs = h_ref[:, :]
    a1 = _dot(hs, w1_ref[0])
    a1 = a1 * jax.nn.sigmoid(a1)
    a2 = _dot(hs, w2_ref[0])
    p = a1 * a2
    mv = jnp.full((128,), jnp.max(p), jnp.float32)
    pw = (p * cw).astype(jnp.bfloat16)
    acc = jnp.where(first, 0.0, out_ref[:, :])
    out_ref[:, :] = acc + _dot(pw, w3_ref[0])

    @pl.when(f == 0)
    def _():
        amax_ref[0, 0, :] = mv

    @pl.when(f != 0)
    def _():
        amax_ref[0, 0, :] = jnp.maximum(amax_ref[0, 0, :], mv)


def kernel(hidden_states, selected_experts, routing_weights, w1, w2, w3):
    sel = selected_experts.astype(jnp.int32)
    hb = hidden_states.astype(jnp.bfloat16)
    out, amax = pl.pallas_call(
        _moe_kernel,
        grid=(_E, _FF // _FT),
        in_specs=[
            pl.BlockSpec((_T, _TOPK), lambda e, f: (0, 0)),
            pl.BlockSpec((_T, _TOPK), lambda e, f: (0, 0)),
            pl.BlockSpec((_T, _H), lambda e, f: (0, 0)),
            pl.BlockSpec((1, _H, _FT), lambda e, f: (e, 0, f)),
            pl.BlockSpec((1, _H, _FT), lambda e, f: (e, 0, f)),
            pl.BlockSpec((1, _FT, _H), lambda e, f: (e, f, 0)),
        ],
        out_specs=[
            pl.BlockSpec((_T, _H), lambda e, f: (0, 0)),
            pl.BlockSpec((1, 1, 128), lambda e, f: (e, 0, 0)),
        ],
        out_shape=[
            jax.ShapeDtypeStruct((_T, _H), jnp.float32),
            jax.ShapeDtypeStruct((_E, 1, 128), jnp.float32),
        ],
    )(sel, routing_weights, hb, w1, w2, w3)
    return out, amax[:, 0, 0]


# speedup vs baseline: 1.0512x; 1.0512x over previous
import functools

import jax
import jax.numpy as jnp
from jax.experimental import pallas as pl
from jax.experimental.pallas import tpu as pltpu

_T = 2048
_H = 1024
_FF = 2048
_E = 8
_TOPK = 2
_TS = 256
_FT = 1024
_NF = _FF // _FT

_dot = functools.partial(
    jax.lax.dot, precision=jax.lax.Precision.DEFAULT,
    preferred_element_type=jnp.float32)


def _moe_kernel(sel_ref, rw_ref, h_ref, w1_ref, w2_ref, w3_ref,
                out_ref, amax_ref, pw_ref):
    e = pl.program_id(0)
    f = pl.program_id(1)

    macc = jnp.full((_FT,), -jnp.inf, jnp.float32)
    for i in range(_T // _TS):
        hs = h_ref[pl.ds(i * _TS, _TS), :]
        a1 = _dot(hs, w1_ref[0])
        a1 = a1 * jax.nn.sigmoid(a1)
        a2 = _dot(hs, w2_ref[0])
        p = a1 * a2
        macc = jnp.maximum(macc, jnp.max(p, axis=0))
        pw_ref[pl.ds(i * _TS, _TS), pl.ds(f * _FT, _FT)] = p.astype(jnp.bfloat16)

    mv = jnp.full((128,), jnp.max(macc), jnp.float32)

    @pl.when(f == 0)
    def _():
        amax_ref[0, 0, :] = mv

    @pl.when(f != 0)
    def _():
        amax_ref[0, 0, :] = jnp.maximum(amax_ref[0, 0, :], mv)

    @pl.when(f == _NF - 1)
    def _():
        cw = (jnp.where(sel_ref[:, 0:1] == e, rw_ref[:, 0:1], 0.0)
              + jnp.where(sel_ref[:, 1:2] == e, rw_ref[:, 1:2], 0.0))
        for i in range(_T // _TS):
            o = _dot(pw_ref[pl.ds(i * _TS, _TS), :], w3_ref[0])
            cw_s = cw[i * _TS:(i + 1) * _TS, :]
            acc = jnp.where(e == 0, 0.0, out_ref[pl.ds(i * _TS, _TS), :])
            out_ref[pl.ds(i * _TS, _TS), :] = acc + o * cw_s


def kernel(hidden_states, selected_experts, routing_weights, w1, w2, w3):
    sel = selected_experts.astype(jnp.int32)
    hb = hidden_states.astype(jnp.bfloat16)
    out, amax = pl.pallas_call(
        _moe_kernel,
        grid=(_E, _NF),
        in_specs=[
            pl.BlockSpec((_T, _TOPK), lambda e, f: (0, 0)),
            pl.BlockSpec((_T, _TOPK), lambda e, f: (0, 0)),
            pl.BlockSpec((_T, _H), lambda e, f: (0, 0)),
            pl.BlockSpec((1, _H, _FT), lambda e, f: (e, 0, f)),
            pl.BlockSpec((1, _H, _FT), lambda e, f: (e, 0, f)),
            pl.BlockSpec((1, _FF, _H), lambda e, f: (e, 0, 0)),
        ],
        out_specs=[
            pl.BlockSpec((_T, _H), lambda e, f: (0, 0)),
            pl.BlockSpec((1, 1, 128), lambda e, f: (e, 0, 0)),
        ],
        out_shape=[
            jax.ShapeDtypeStruct((_T, _H), jnp.float32),
            jax.ShapeDtypeStruct((_E, 1, 128), jnp.float32),
        ],
        scratch_shapes=[pltpu.VMEM((_T, _FF), jnp.bfloat16)],
    )(sel, routing_weights, hb, w1, w2, w3)
    return out, amax[:, 0, 0]


# speedup vs baseline: 1.0529x; 1.0016x over previous
import functools

import jax
import jax.numpy as jnp
from jax.experimental import pallas as pl
from jax.experimental.pallas import tpu as pltpu

_T = 2048
_H = 1024
_FF = 2048
_E = 8
_TOPK = 2
_TS = 256
_TS2 = 256
_FT = 1024
_NF = _FF // _FT

_dot = functools.partial(
    jax.lax.dot, precision=jax.lax.Precision.DEFAULT,
    preferred_element_type=jnp.float32)


def _moe_kernel(sel_ref, rw_ref, h_ref, w1_ref, w2_ref, w3_ref,
                out_ref, amax_ref, pw_ref):
    e = pl.program_id(0)
    f = pl.program_id(1)

    macc = jnp.full((_FT,), -jnp.inf, jnp.float32)
    for i in range(_T // _TS):
        hs = h_ref[pl.ds(i * _TS, _TS), :]
        a1 = _dot(hs, w1_ref[0])
        a1 = a1 * jax.nn.sigmoid(a1)
        a2 = _dot(hs, w2_ref[0])
        p = a1 * a2
        macc = jnp.maximum(macc, jnp.max(p, axis=0))
        pw_ref[pl.ds(i * _TS, _TS), pl.ds(f * _FT, _FT)] = p.astype(jnp.bfloat16)

    mv = jnp.full((128,), jnp.max(macc), jnp.float32)

    @pl.when(f == 0)
    def _():
        amax_ref[0, 0, :] = mv

    @pl.when(f != 0)
    def _():
        amax_ref[0, 0, :] = jnp.maximum(amax_ref[0, 0, :], mv)

    @pl.when(f == _NF - 1)
    def _():
        cw = (jnp.where(sel_ref[:, 0:1] == e, rw_ref[:, 0:1], 0.0)
              + jnp.where(sel_ref[:, 1:2] == e, rw_ref[:, 1:2], 0.0))
        for i in range(_T // _TS2):
            o = _dot(pw_ref[pl.ds(i * _TS2, _TS2), :], w3_ref[0])
            cw_s = cw[i * _TS2:(i + 1) * _TS2, :]
            acc = jnp.where(e == 0, 0.0, out_ref[pl.ds(i * _TS2, _TS2), :])
            out_ref[pl.ds(i * _TS2, _TS2), :] = acc + o * cw_s


def kernel(hidden_states, selected_experts, routing_weights, w1, w2, w3):
    sel = selected_experts.astype(jnp.int32)
    hb = hidden_states.astype(jnp.bfloat16)
    out, amax = pl.pallas_call(
        _moe_kernel,
        grid=(_E, _NF),
        in_specs=[
            pl.BlockSpec((_T, _TOPK), lambda e, f: (0, 0)),
            pl.BlockSpec((_T, _TOPK), lambda e, f: (0, 0)),
            pl.BlockSpec((_T, _H), lambda e, f: (0, 0)),
            pl.BlockSpec((1, _H, _FT), lambda e, f: (e, 0, f)),
            pl.BlockSpec((1, _H, _FT), lambda e, f: (e, 0, f)),
            pl.BlockSpec((1, _FF, _H), lambda e, f: (e, 0, 0)),
        ],
        out_specs=[
            pl.BlockSpec((_T, _H), lambda e, f: (0, 0)),
            pl.BlockSpec((1, 1, 128), lambda e, f: (e, 0, 0)),
        ],
        out_shape=[
            jax.ShapeDtypeStruct((_T, _H), jnp.float32),
            jax.ShapeDtypeStruct((_E, 1, 128), jnp.float32),
        ],
        scratch_shapes=[pltpu.VMEM((_T, _FF), jnp.bfloat16)],
    )(sel, routing_weights, hb, w1, w2, w3)
    return out, amax[:, 0, 0]


# speedup vs baseline: 1.0560x; 1.0029x over previous
import functools

import jax
import jax.numpy as jnp
from jax.experimental import pallas as pl
from jax.experimental.pallas import tpu as pltpu

_T = 2048
_H = 1024
_FF = 2048
_E = 8
_TOPK = 2
_TS = 256
_TS2 = 512
_FT = 1024
_NF = _FF // _FT

_dot = functools.partial(
    jax.lax.dot, precision=jax.lax.Precision.DEFAULT,
    preferred_element_type=jnp.float32)


def _moe_kernel(sel_ref, rw_ref, h_ref, w1_ref, w2_ref, w3_ref,
                out_ref, amax_ref, pw_ref):
    e = pl.program_id(0)
    f = pl.program_id(1)

    macc = jnp.full((_FT,), -jnp.inf, jnp.float32)
    for i in range(_T // _TS):
        hs = h_ref[pl.ds(i * _TS, _TS), :]
        a1 = _dot(hs, w1_ref[0])
        a1 = a1 * jax.nn.sigmoid(a1)
        a2 = _dot(hs, w2_ref[0])
        p = a1 * a2
        macc = jnp.maximum(macc, jnp.max(p, axis=0))
        pw_ref[pl.ds(i * _TS, _TS), pl.ds(f * _FT, _FT)] = p.astype(jnp.bfloat16)

    mv = jnp.full((128,), jnp.max(macc), jnp.float32)

    @pl.when(f == 0)
    def _():
        amax_ref[0, 0, :] = mv

    @pl.when(f != 0)
    def _():
        amax_ref[0, 0, :] = jnp.maximum(amax_ref[0, 0, :], mv)

    @pl.when(f == _NF - 1)
    def _():
        cw = (jnp.where(sel_ref[:, 0:1] == e, rw_ref[:, 0:1], 0.0)
              + jnp.where(sel_ref[:, 1:2] == e, rw_ref[:, 1:2], 0.0))
        for i in range(_T // _TS2):
            o = _dot(pw_ref[pl.ds(i * _TS2, _TS2), :], w3_ref[0])
            cw_s = cw[i * _TS2:(i + 1) * _TS2, :]
            acc = jnp.where(e == 0, 0.0, out_ref[pl.ds(i * _TS2, _TS2), :])
            out_ref[pl.ds(i * _TS2, _TS2), :] = acc + o * cw_s


def kernel(hidden_states, selected_experts, routing_weights, w1, w2, w3):
    sel = selected_experts.astype(jnp.int32)
    hb = hidden_states.astype(jnp.bfloat16)
    out, amax = pl.pallas_call(
        _moe_kernel,
        grid=(_E, _NF),
        in_specs=[
            pl.BlockSpec((_T, _TOPK), lambda e, f: (0, 0)),
            pl.BlockSpec((_T, _TOPK), lambda e, f: (0, 0)),
            pl.BlockSpec((_T, _H), lambda e, f: (0, 0)),
            pl.BlockSpec((1, _H, _FT), lambda e, f: (e, 0, f)),
            pl.BlockSpec((1, _H, _FT), lambda e, f: (e, 0, f)),
            pl.BlockSpec((1, _FF, _H), lambda e, f: (e, 0, 0)),
        ],
        out_specs=[
            pl.BlockSpec((_T, _H), lambda e, f: (0, 0)),
            pl.BlockSpec((1, 1, 128), lambda e, f: (e, 0, 0)),
        ],
        out_shape=[
            jax.ShapeDtypeStruct((_T, _H), jnp.float32),
            jax.ShapeDtypeStruct((_E, 1, 128), jnp.float32),
        ],
        scratch_shapes=[pltpu.VMEM((_T, _FF), jnp.bfloat16)],
        compiler_params=pltpu.CompilerParams(
            vmem_limit_bytes=63 * 1024 * 1024),
    )(sel, routing_weights, hb, w1, w2, w3)
    return out, amax[:, 0, 0]
